# baseline (speedup 1.0000x reference)
import functools

import jax
import jax.numpy as jnp
from jax import lax
from jax.experimental import pallas as pl
from jax.experimental.pallas import tpu as pltpu
from jax.experimental.pallas import tpu_sc as plsc

_B = 16384
_F = 26
_D = 16
_N = _B * _F
_IN = _F * _D
_H1 = 128
_H2 = 64
_EPS = 1e-5

_NW = 32
_RPW = _N // _NW
_CH = 128
_NCH = _RPW // _CH
_NB = 8
_NGRP = _NCH // _NB

_BC = 512
_NC = _B // _BC


_SPW = _B // _NW


def _sc_gather_body(idx_hbm, emb_hbm, fc_hbm, out_e, out_lin,
                    idx_v, ebuf, fbuf, lin_v, gsem, fsem, wsem):
    wid = lax.axis_index("s") * 2 + lax.axis_index("c")
    pltpu.sync_copy(idx_hbm.at[wid], idx_v)
    base = wid * _RPW

    def group(g, carry):
        j0 = g * _NB
        gds = []
        for b in range(_NB):
            row_idx = idx_v.at[j0 + b]
            gds.append(pltpu.async_copy(emb_hbm.at[row_idx], ebuf.at[b], gsem))
            gds.append(pltpu.async_copy(fc_hbm.at[row_idx], fbuf.at[j0 + b], fsem))
        for d in gds:
            d.wait()
        wds = []
        for b in range(_NB):
            row0 = base + (j0 + b) * _CH
            wds.append(pltpu.async_copy(ebuf.at[b], out_e.at[pl.ds(row0, _CH)], wsem))
        for d in wds:
            d.wait()
        return carry

    lax.fori_loop(0, _NGRP, group, 0)

    lane = lax.iota(jnp.int32, 16)

    def lin_step(g, carry):
        flat0 = (g * 16 + lane) * _F
        acc = jnp.zeros((16,), jnp.float32)
        for o in range(_F):
            fp = flat0 + o
            acc = acc + plsc.load_gather(fbuf, [fp >> 7, fp & 127])
        lin_v[pl.ds(g * 16, 16)] = acc
        return carry

    lax.fori_loop(0, _SPW // 16, lin_step, 0)
    pltpu.sync_copy(lin_v, out_lin.at[pl.ds(wid * _SPW, _SPW)])


_sc_gather = functools.partial(
    pl.kernel,
    out_type=(jax.ShapeDtypeStruct((_N, _D), jnp.bfloat16),
              jax.ShapeDtypeStruct((_B,), jnp.float32)),
    mesh=plsc.VectorSubcoreMesh(core_axis_name="c", subcore_axis_name="s"),
    scratch_types=[
        pltpu.VMEM((_NCH, _CH), jnp.int32),
        pltpu.VMEM((_NB, _CH, _D), jnp.bfloat16),
        pltpu.VMEM((_NCH, _CH), jnp.float32),
        pltpu.VMEM((_SPW,), jnp.float32),
        pltpu.SemaphoreType.DMA,
        pltpu.SemaphoreType.DMA,
        pltpu.SemaphoreType.DMA,
    ],
    compiler_params=pltpu.CompilerParams(use_tc_tiling_on_sc=False,
                                         needs_layout_passes=False),
)(_sc_gather_body)


def _tc_body(emb_ref, lin_ref, w1_ref, b1_ref, g1_ref, be1_ref,
             w2_ref, b2_ref, g2_ref, be2_ref, w3_ref, cst_ref, m_ref,
             out_ref, h1_s, h2_s, s1_s, q1_s, s2_s, q2_s, base_s):
    p = pl.program_id(0)
    i = pl.program_id(1)
    row0 = i * _BC

    @pl.when(p == 0)
    def _phase0():
        e = emb_ref[...]
        h1 = lax.dot_general(e, w1_ref[...], (((1,), (1,)), ((), ())),
                             preferred_element_type=jnp.float32)
        h1 = h1 + b1_ref[...]
        h1_s[pl.ds(row0, _BC), :] = h1

        @pl.when(i == 0)
        def _():
            s1_s[...] = jnp.zeros_like(s1_s)
            q1_s[...] = jnp.zeros_like(q1_s)

        s1_s[...] += jnp.sum(h1, axis=0, keepdims=True)
        q1_s[...] += jnp.sum(h1 * h1, axis=0, keepdims=True)

        t = lax.dot_general(e, m_ref[...], (((1,), (0,)), ((), ())),
                            preferred_element_type=jnp.float32)
        ef = e.astype(jnp.float32)
        fm = 0.5 * (jnp.sum(t * t, axis=1, keepdims=True)
                    - jnp.sum(ef * ef, axis=1, keepdims=True))
        base_s[pl.ds(row0, _BC), :] = fm

    @pl.when(p == 1)
    def _phase1():
        mean = s1_s[...] * (1.0 / _B)
        var = q1_s[...] * (1.0 / _B) - mean * mean
        scale = lax.rsqrt(var + _EPS) * g1_ref[...]
        h1 = h1_s[pl.ds(row0, _BC), :]
        a = jnp.maximum((h1 - mean) * scale + be1_ref[...], 0.0)
        h2 = lax.dot_general(a, w2_ref[...], (((1,), (1,)), ((), ())),
                             preferred_element_type=jnp.float32)
        h2 = h2 + b2_ref[...]
        h2_s[pl.ds(row0, _BC), :] = h2

        @pl.when(i == 0)
        def _():
            s2_s[...] = jnp.zeros_like(s2_s)
            q2_s[...] = jnp.zeros_like(q2_s)

        s2_s[...] += jnp.sum(h2, axis=0, keepdims=True)
        q2_s[...] += jnp.sum(h2 * h2, axis=0, keepdims=True)

    @pl.when(p == 2)
    def _phase2():
        mean = s2_s[...] * (1.0 / _B)
        var = q2_s[...] * (1.0 / _B) - mean * mean
        scale = lax.rsqrt(var + _EPS) * g2_ref[...]
        h2 = h2_s[pl.ds(row0, _BC), :]
        a = jnp.maximum((h2 - mean) * scale + be2_ref[...], 0.0)
        mlp = jnp.sum(a * w3_ref[...], axis=1, keepdims=True)
        z = base_s[pl.ds(row0, _BC), :] + lin_ref[...] + mlp + cst_ref[0]
        out_ref[...] = jax.nn.sigmoid(z)


def _tc_forward(e2, lin2, w1, b1, g1, be1, w2, b2, g2, be2, w3, cst, m):
    whole = lambda shape: pl.BlockSpec(shape, lambda p, i: (0,) * len(shape))
    return pl.pallas_call(
        _tc_body,
        grid=(3, _NC),
        in_specs=[
            pl.BlockSpec((_BC, _IN), lambda p, i: (jnp.where(p == 0, i, 0), 0)),
            pl.BlockSpec((_BC, 1), lambda p, i: (jnp.where(p == 2, i, 0), 0)),
            whole((_H1, _IN)),
            whole((1, _H1)),
            whole((1, _H1)),
            whole((1, _H1)),
            whole((_H2, _H1)),
            whole((1, _H2)),
            whole((1, _H2)),
            whole((1, _H2)),
            whole((1, _H2)),
            pl.BlockSpec(memory_space=pltpu.SMEM),
            whole((_IN, _D)),
        ],
        out_specs=pl.BlockSpec((_BC, 1), lambda p, i: (i, 0)),
        out_shape=jax.ShapeDtypeStruct((_B, 1), jnp.float32),
        scratch_shapes=[
            pltpu.VMEM((_B, _H1), jnp.float32),
            pltpu.VMEM((_B, _H2), jnp.float32),
            pltpu.VMEM((1, _H1), jnp.float32),
            pltpu.VMEM((1, _H1), jnp.float32),
            pltpu.VMEM((1, _H2), jnp.float32),
            pltpu.VMEM((1, _H2), jnp.float32),
            pltpu.VMEM((_B, 1), jnp.float32),
        ],
    )(e2, lin2, w1, b1, g1, be1, w2, b2, g2, be2, w3, cst, m)


def kernel(x, offsets, emb, fc_w, fc_b, W1, b1, g1, be1, W2, b2, g2, be2, W3, b3):
    idx = (x + offsets[None, :]).reshape(_NW, _NCH, _CH)
    emb_rows, lin = _sc_gather(idx, emb.astype(jnp.bfloat16), fc_w.T.reshape(-1))
    e2 = emb_rows.reshape(_B, _IN)
    lin2 = lin.reshape(_B, 1)
    cst = (fc_b + b3).reshape(1)
    m = jnp.tile(jnp.eye(_D, dtype=jnp.bfloat16), (_F, 1))
    out = _tc_forward(e2, lin2, W1.astype(jnp.bfloat16),
                      b1.reshape(1, _H1), g1.reshape(1, _H1), be1.reshape(1, _H1),
                      W2, b2.reshape(1, _H2), g2.reshape(1, _H2), be2.reshape(1, _H2),
                      W3, cst, m)
    return out.reshape(_B)


# speedup vs baseline: 1.0572x; 1.0572x over previous
import functools

import jax
import jax.numpy as jnp
from jax import lax
from jax.experimental import pallas as pl
from jax.experimental.pallas import tpu as pltpu
from jax.experimental.pallas import tpu_sc as plsc

_B = 16384
_F = 26
_D = 16
_N = _B * _F
_IN = _F * _D
_H1 = 128
_H2 = 64
_EPS = 1e-5

_NW = 32
_RPW = _N // _NW
_CH = 128
_NCH = _RPW // _CH
_NB = 8
_NGRP = _NCH // _NB

_BC = 512
_NC = _B // _BC


_SPW = _B // _NW


def _sc_gather_body(idx_hbm, emb_hbm, fc_hbm, out_e, out_lin,
                    idx_v, ebuf, fbuf, lin_v, gsem, fsem, wsem):
    wid = lax.axis_index("s") * 2 + lax.axis_index("c")
    pltpu.sync_copy(idx_hbm.at[wid], idx_v)
    base = wid * _RPW

    def group(g, carry):
        j0 = g * _NB
        gds = []
        for b in range(_NB):
            row_idx = idx_v.at[j0 + b]
            gds.append(pltpu.async_copy(emb_hbm.at[row_idx], ebuf.at[b], gsem))
            gds.append(pltpu.async_copy(fc_hbm.at[row_idx], fbuf.at[j0 + b], fsem))
        for d in gds:
            d.wait()
        wds = []
        for b in range(_NB):
            row0 = base + (j0 + b) * _CH
            wds.append(pltpu.async_copy(ebuf.at[b], out_e.at[pl.ds(row0, _CH)], wsem))
        for d in wds:
            d.wait()
        return carry

    lax.fori_loop(0, _NGRP, group, 0)

    lane = lax.iota(jnp.int32, 16)

    def lin_step(g, carry):
        flat0 = (g * 16 + lane) * _F
        acc = jnp.zeros((16,), jnp.float32)
        for o in range(_F):
            fp = flat0 + o
            acc = acc + plsc.load_gather(fbuf, [fp >> 7, fp & 127])
        lin_v[pl.ds(g * 16, 16)] = acc
        return carry

    lax.fori_loop(0, _SPW // 16, lin_step, 0)
    pltpu.sync_copy(lin_v, out_lin.at[pl.ds(wid * _SPW, _SPW)])


_sc_gather = functools.partial(
    pl.kernel,
    out_type=(jax.ShapeDtypeStruct((_N, _D), jnp.float32),
              jax.ShapeDtypeStruct((_B,), jnp.float32)),
    mesh=plsc.VectorSubcoreMesh(core_axis_name="c", subcore_axis_name="s"),
    scratch_types=[
        pltpu.VMEM((_NCH, _CH), jnp.int32),
        pltpu.VMEM((_NB, _CH, _D), jnp.float32),
        pltpu.VMEM((_NCH, _CH), jnp.float32),
        pltpu.VMEM((_SPW,), jnp.float32),
        pltpu.SemaphoreType.DMA,
        pltpu.SemaphoreType.DMA,
        pltpu.SemaphoreType.DMA,
    ],
    compiler_params=pltpu.CompilerParams(use_tc_tiling_on_sc=False,
                                         needs_layout_passes=False),
)(_sc_gather_body)


def _tc_body(emb_ref, lin_ref, w1_ref, b1_ref, g1_ref, be1_ref,
             w2_ref, b2_ref, g2_ref, be2_ref, w3_ref, cst_ref, m_ref,
             out_ref, h1_s, h2_s, s1_s, q1_s, s2_s, q2_s, base_s):
    p = pl.program_id(0)
    i = pl.program_id(1)
    row0 = i * _BC

    @pl.when(p == 0)
    def _phase0():
        e = emb_ref[...]
        h1 = lax.dot_general(e, w1_ref[...], (((1,), (1,)), ((), ())),
                             preferred_element_type=jnp.float32)
        h1 = h1 + b1_ref[...]
        h1_s[pl.ds(row0, _BC), :] = h1

        @pl.when(i == 0)
        def _():
            s1_s[...] = jnp.zeros_like(s1_s)
            q1_s[...] = jnp.zeros_like(q1_s)

        s1_s[...] += jnp.sum(h1, axis=0, keepdims=True)
        q1_s[...] += jnp.sum(h1 * h1, axis=0, keepdims=True)

        t = lax.dot_general(e, m_ref[...], (((1,), (0,)), ((), ())),
                            preferred_element_type=jnp.float32)
        ef = e.astype(jnp.float32)
        fm = 0.5 * (jnp.sum(t * t, axis=1, keepdims=True)
                    - jnp.sum(ef * ef, axis=1, keepdims=True))
        base_s[pl.ds(row0, _BC), :] = fm

    @pl.when(p == 1)
    def _phase1():
        mean = s1_s[...] * (1.0 / _B)
        var = q1_s[...] * (1.0 / _B) - mean * mean
        scale = lax.rsqrt(var + _EPS) * g1_ref[...]
        h1 = h1_s[pl.ds(row0, _BC), :]
        a = jnp.maximum((h1 - mean) * scale + be1_ref[...], 0.0)
        h2 = lax.dot_general(a, w2_ref[...], (((1,), (1,)), ((), ())),
                             preferred_element_type=jnp.float32)
        h2 = h2 + b2_ref[...]
        h2_s[pl.ds(row0, _BC), :] = h2

        @pl.when(i == 0)
        def _():
            s2_s[...] = jnp.zeros_like(s2_s)
            q2_s[...] = jnp.zeros_like(q2_s)

        s2_s[...] += jnp.sum(h2, axis=0, keepdims=True)
        q2_s[...] += jnp.sum(h2 * h2, axis=0, keepdims=True)

    @pl.when(p == 2)
    def _phase2():
        mean = s2_s[...] * (1.0 / _B)
        var = q2_s[...] * (1.0 / _B) - mean * mean
        scale = lax.rsqrt(var + _EPS) * g2_ref[...]
        h2 = h2_s[pl.ds(row0, _BC), :]
        a = jnp.maximum((h2 - mean) * scale + be2_ref[...], 0.0)
        mlp = jnp.sum(a * w3_ref[...], axis=1, keepdims=True)
        z = base_s[pl.ds(row0, _BC), :] + lin_ref[...] + mlp + cst_ref[0]
        out_ref[...] = jax.nn.sigmoid(z)


def _tc_forward(e2, lin2, w1, b1, g1, be1, w2, b2, g2, be2, w3, cst, m):
    whole = lambda shape: pl.BlockSpec(shape, lambda p, i: (0,) * len(shape))
    return pl.pallas_call(
        _tc_body,
        grid=(3, _NC),
        in_specs=[
            pl.BlockSpec((_BC, _IN), lambda p, i: (jnp.where(p == 0, i, 0), 0)),
            pl.BlockSpec((_BC, 1), lambda p, i: (jnp.where(p == 2, i, 0), 0)),
            whole((_H1, _IN)),
            whole((1, _H1)),
            whole((1, _H1)),
            whole((1, _H1)),
            whole((_H2, _H1)),
            whole((1, _H2)),
            whole((1, _H2)),
            whole((1, _H2)),
            whole((1, _H2)),
            pl.BlockSpec(memory_space=pltpu.SMEM),
            whole((_IN, _D)),
        ],
        out_specs=pl.BlockSpec((_BC, 1), lambda p, i: (i, 0)),
        out_shape=jax.ShapeDtypeStruct((_B, 1), jnp.float32),
        scratch_shapes=[
            pltpu.VMEM((_B, _H1), jnp.float32),
            pltpu.VMEM((_B, _H2), jnp.float32),
            pltpu.VMEM((1, _H1), jnp.float32),
            pltpu.VMEM((1, _H1), jnp.float32),
            pltpu.VMEM((1, _H2), jnp.float32),
            pltpu.VMEM((1, _H2), jnp.float32),
            pltpu.VMEM((_B, 1), jnp.float32),
        ],
    )(e2, lin2, w1, b1, g1, be1, w2, b2, g2, be2, w3, cst, m)


_DT_BR = 8192
_DT_G = (999986 + _DT_BR - 1) // _DT_BR


def _detile_body(src_ref, dst_ref):
    for s in range(8):
        dst_ref[:, s * _D:(s + 1) * _D] = src_ref[pl.ds(s, _DT_BR // 8, 8), :]


_detile = pl.pallas_call(
    _detile_body,
    grid=(_DT_G,),
    in_specs=[pl.BlockSpec((_DT_BR, _D), lambda i: (i, 0))],
    out_specs=pl.BlockSpec((_DT_BR // 8, 128), lambda i: (i, 0)),
    out_shape=jax.ShapeDtypeStruct((999992 // 8, 128), jnp.float32),
)


def kernel(x, offsets, emb, fc_w, fc_b, W1, b1, g1, be1, W2, b2, g2, be2, W3, b3):
    idx = (x + offsets[None, :]).reshape(_NW, _NCH, _CH)
    emb_lin = _detile(emb).reshape(999992, _D)
    emb_rows, lin = _sc_gather(idx, emb_lin, fc_w.reshape(-1))
    e2 = emb_rows.reshape(_B, _IN)
    lin2 = lin.reshape(_B, 1)
    cst = (fc_b + b3).reshape(1)
    m = jnp.tile(jnp.eye(_D, dtype=jnp.float32), (_F, 1))
    out = _tc_forward(e2, lin2, W1,
                      b1.reshape(1, _H1), g1.reshape(1, _H1), be1.reshape(1, _H1),
                      W2, b2.reshape(1, _H2), g2.reshape(1, _H2), be2.reshape(1, _H2),
                      W3, cst, m)
    return out.reshape(_B)


# speedup vs baseline: 1.1959x; 1.1312x over previous
import functools

import jax
import jax.numpy as jnp
from jax import lax
from jax.experimental import pallas as pl
from jax.experimental.pallas import tpu as pltpu
from jax.experimental.pallas import tpu_sc as plsc

_B = 16384
_F = 26
_D = 16
_N = _B * _F
_IN = _F * _D
_H1 = 128
_H2 = 64
_EPS = 1e-5

_NW = 32
_RPW = _N // _NW
_CH = 128
_NCH = _RPW // _CH
_NB = 8
_NGRP = _NCH // _NB

_BC = 512
_NC = _B // _BC


_SPW = _B // _NW


def _sc_gather_body(idx_hbm, emb_hbm, fc_hbm, out_e, out_lin,
                    idx_v, ebuf, fbuf, lin_v, gsem, fsem, wsem):
    wid = lax.axis_index("s") * 2 + lax.axis_index("c")
    pltpu.sync_copy(idx_hbm.at[wid], idx_v)
    base = wid * _RPW

    half = _NB // 2
    ngrp2 = _NCH // half

    def fire_gathers(j0, ring):
        ds_ = []
        for b in range(half):
            row_idx = idx_v.at[j0 + b]
            ds_.append(pltpu.async_copy(emb_hbm.at[row_idx],
                                        ebuf.at[ring * half + b], gsem))
            ds_.append(pltpu.async_copy(fc_hbm.at[row_idx], fbuf.at[j0 + b], fsem))
        return ds_

    def fire_writes(j0, ring):
        ds_ = []
        for b in range(half):
            row0 = base + (j0 + b) * _CH
            ds_.append(pltpu.async_copy(ebuf.at[ring * half + b],
                                        out_e.at[pl.ds(row0, _CH)], wsem))
        return ds_

    for d in fire_gathers(0, 0):
        d.wait()
    fire_writes(0, 0)

    def group(g, carry):
        ring = g % 2
        j0 = g * half
        gds = fire_gathers(j0, ring)
        for d in gds:
            d.wait()
        wds = fire_writes(j0, ring)
        for d in wds:
            d.wait()
        return carry

    lax.fori_loop(1, ngrp2, group, 0)
    for b in range(half):
        row0 = base + (_NCH - half + b) * _CH
        pltpu.make_async_copy(ebuf.at[b], out_e.at[pl.ds(row0, _CH)], wsem).wait()

    lane = lax.iota(jnp.int32, 16)

    def lin_step(g, carry):
        flat0 = (g * 16 + lane) * _F
        acc = jnp.zeros((16,), jnp.float32)
        for o in range(_F):
            fp = flat0 + o
            acc = acc + plsc.load_gather(fbuf, [fp >> 7, fp & 127])
        lin_v[pl.ds(g * 16, 16)] = acc
        return carry

    lax.fori_loop(0, _SPW // 16, lin_step, 0)
    pltpu.sync_copy(lin_v, out_lin.at[pl.ds(wid * _SPW, _SPW)])


_sc_gather = functools.partial(
    pl.kernel,
    out_type=(jax.ShapeDtypeStruct((_N, _D), jnp.float32),
              jax.ShapeDtypeStruct((_B,), jnp.float32)),
    mesh=plsc.VectorSubcoreMesh(core_axis_name="c", subcore_axis_name="s"),
    scratch_types=[
        pltpu.VMEM((_NCH, _CH), jnp.int32),
        pltpu.VMEM((_NB, _CH, _D), jnp.float32),
        pltpu.VMEM((_NCH, _CH), jnp.float32),
        pltpu.VMEM((_SPW,), jnp.float32),
        pltpu.SemaphoreType.DMA,
        pltpu.SemaphoreType.DMA,
        pltpu.SemaphoreType.DMA,
    ],
    compiler_params=pltpu.CompilerParams(use_tc_tiling_on_sc=False,
                                         needs_layout_passes=False),
)(_sc_gather_body)


def _tc_body(emb_ref, lin_ref, w1_ref, b1_ref, g1_ref, be1_ref,
             w2_ref, b2_ref, g2_ref, be2_ref, w3_ref, cst_ref, m_ref,
             out_ref, h1_s, h2_s, s1_s, q1_s, s2_s, q2_s, base_s):
    p = pl.program_id(0)
    i = pl.program_id(1)
    row0 = i * _BC

    @pl.when(p == 0)
    def _phase0():
        e = emb_ref[...]
        eb = e.astype(jnp.bfloat16)
        h1 = lax.dot_general(eb, w1_ref[...], (((1,), (1,)), ((), ())),
                             preferred_element_type=jnp.float32)
        h1 = h1 + b1_ref[...]
        h1_s[pl.ds(row0, _BC), :] = h1

        @pl.when(i == 0)
        def _():
            s1_s[...] = jnp.zeros_like(s1_s)
            q1_s[...] = jnp.zeros_like(q1_s)

        s1_s[...] += jnp.sum(h1, axis=0, keepdims=True)
        q1_s[...] += jnp.sum(h1 * h1, axis=0, keepdims=True)

        t = lax.dot_general(eb, m_ref[...], (((1,), (0,)), ((), ())),
                            preferred_element_type=jnp.float32)
        fm = 0.5 * (jnp.sum(t * t, axis=1, keepdims=True)
                    - jnp.sum(e * e, axis=1, keepdims=True))
        base_s[pl.ds(row0, _BC), :] = fm

    @pl.when(p == 1)
    def _phase1():
        mean = s1_s[...] * (1.0 / _B)
        var = q1_s[...] * (1.0 / _B) - mean * mean
        scale = lax.rsqrt(var + _EPS) * g1_ref[...]
        h1 = h1_s[pl.ds(row0, _BC), :]
        a = jnp.maximum((h1 - mean) * scale + be1_ref[...], 0.0)
        h2 = lax.dot_general(a, w2_ref[...], (((1,), (1,)), ((), ())),
                             preferred_element_type=jnp.float32)
        h2 = h2 + b2_ref[...]
        h2_s[pl.ds(row0, _BC), :] = h2

        @pl.when(i == 0)
        def _():
            s2_s[...] = jnp.zeros_like(s2_s)
            q2_s[...] = jnp.zeros_like(q2_s)

        s2_s[...] += jnp.sum(h2, axis=0, keepdims=True)
        q2_s[...] += jnp.sum(h2 * h2, axis=0, keepdims=True)

    @pl.when(p == 2)
    def _phase2():
        mean = s2_s[...] * (1.0 / _B)
        var = q2_s[...] * (1.0 / _B) - mean * mean
        scale = lax.rsqrt(var + _EPS) * g2_ref[...]
        h2 = h2_s[pl.ds(row0, _BC), :]
        a = jnp.maximum((h2 - mean) * scale + be2_ref[...], 0.0)
        mlp = jnp.sum(a * w3_ref[...], axis=1, keepdims=True)
        z = base_s[pl.ds(row0, _BC), :] + lin_ref[...] + mlp + cst_ref[0]
        out_ref[...] = jax.nn.sigmoid(z)


def _tc_forward(e2, lin2, w1, b1, g1, be1, w2, b2, g2, be2, w3, cst, m):
    whole = lambda shape: pl.BlockSpec(shape, lambda p, i: (0,) * len(shape))
    return pl.pallas_call(
        _tc_body,
        grid=(3, _NC),
        in_specs=[
            pl.BlockSpec((_BC, _IN), lambda p, i: (jnp.where(p == 0, i, 0), 0)),
            pl.BlockSpec((_BC, 1), lambda p, i: (jnp.where(p == 2, i, 0), 0)),
            whole((_H1, _IN)),
            whole((1, _H1)),
            whole((1, _H1)),
            whole((1, _H1)),
            whole((_H2, _H1)),
            whole((1, _H2)),
            whole((1, _H2)),
            whole((1, _H2)),
            whole((1, _H2)),
            pl.BlockSpec(memory_space=pltpu.SMEM),
            whole((_IN, _D)),
        ],
        out_specs=pl.BlockSpec((_BC, 1), lambda p, i: (i, 0)),
        out_shape=jax.ShapeDtypeStruct((_B, 1), jnp.float32),
        scratch_shapes=[
            pltpu.VMEM((_B, _H1), jnp.float32),
            pltpu.VMEM((_B, _H2), jnp.float32),
            pltpu.VMEM((1, _H1), jnp.float32),
            pltpu.VMEM((1, _H1), jnp.float32),
            pltpu.VMEM((1, _H2), jnp.float32),
            pltpu.VMEM((1, _H2), jnp.float32),
            pltpu.VMEM((_B, 1), jnp.float32),
        ],
    )(e2, lin2, w1, b1, g1, be1, w2, b2, g2, be2, w3, cst, m)


def kernel(x, offsets, emb, fc_w, fc_b, W1, b1, g1, be1, W2, b2, g2, be2, W3, b3):
    idx = (x + offsets[None, :]).reshape(_NW, _NCH, _CH)
    emb_rows, lin = _sc_gather(idx, emb, lax.reshape(fc_w, (999986,), dimensions=(1, 0)))
    e2 = emb_rows.reshape(_B, _IN)
    lin2 = lin.reshape(_B, 1)
    cst = (fc_b + b3).reshape(1)
    m = jnp.tile(jnp.eye(_D, dtype=jnp.bfloat16), (_F, 1))
    out = _tc_forward(e2, lin2, W1.astype(jnp.bfloat16),
                      b1.reshape(1, _H1), g1.reshape(1, _H1), be1.reshape(1, _H1),
                      W2, b2.reshape(1, _H2), g2.reshape(1, _H2), be2.reshape(1, _H2),
                      W3, cst, m)
    return out.reshape(_B)


# speedup vs baseline: 1.2105x; 1.0122x over previous
import functools

import jax
import jax.numpy as jnp
from jax import lax
from jax.experimental import pallas as pl
from jax.experimental.pallas import tpu as pltpu
from jax.experimental.pallas import tpu_sc as plsc

_B = 16384
_F = 26
_D = 16
_N = _B * _F
_IN = _F * _D
_H1 = 128
_H2 = 64
_EPS = 1e-5

_NW = 32
_RPW = _N // _NW
_CH = 128
_NCH = _RPW // _CH
_NB = 16
_NGRP = _NCH // _NB

_BC = 512
_NC = _B // _BC


_SPW = _B // _NW


def _sc_gather_body(idx_hbm, emb_hbm, fc_hbm, out_e, out_lin,
                    idx_v, ebuf, fbuf, lin_v, gsem, fsem, wsem):
    wid = lax.axis_index("s") * 2 + lax.axis_index("c")
    pltpu.sync_copy(idx_hbm.at[wid], idx_v)
    base = wid * _RPW

    half = _NB // 2
    ngrp2 = _NCH // half

    def fire_gathers(j0, ring):
        ds_ = []
        for b in range(half):
            row_idx = idx_v.at[j0 + b]
            ds_.append(pltpu.async_copy(emb_hbm.at[row_idx],
                                        ebuf.at[ring * half + b], gsem))
            ds_.append(pltpu.async_copy(fc_hbm.at[row_idx], fbuf.at[j0 + b], fsem))
        return ds_

    def fire_writes(j0, ring):
        ds_ = []
        for b in range(half):
            row0 = base + (j0 + b) * _CH
            ds_.append(pltpu.async_copy(ebuf.at[ring * half + b],
                                        out_e.at[pl.ds(row0, _CH)], wsem))
        return ds_

    for d in fire_gathers(0, 0):
        d.wait()
    fire_writes(0, 0)

    def group(g, carry):
        ring = g % 2
        j0 = g * half
        gds = fire_gathers(j0, ring)
        for d in gds:
            d.wait()
        wds = fire_writes(j0, ring)
        for d in wds:
            d.wait()
        return carry

    lax.fori_loop(1, ngrp2, group, 0)
    for b in range(half):
        row0 = base + (_NCH - half + b) * _CH
        pltpu.make_async_copy(ebuf.at[b], out_e.at[pl.ds(row0, _CH)], wsem).wait()

    lane = lax.iota(jnp.int32, 16)

    def lin_step(g, carry):
        flat0 = (g * 16 + lane) * _F
        acc = jnp.zeros((16,), jnp.float32)
        for o in range(_F):
            fp = flat0 + o
            acc = acc + plsc.load_gather(fbuf, [fp >> 7, fp & 127])
        lin_v[pl.ds(g * 16, 16)] = acc
        return carry

    lax.fori_loop(0, _SPW // 16, lin_step, 0)
    pltpu.sync_copy(lin_v, out_lin.at[pl.ds(wid * _SPW, _SPW)])


_sc_gather = functools.partial(
    pl.kernel,
    out_type=(jax.ShapeDtypeStruct((_N, _D), jnp.float32),
              jax.ShapeDtypeStruct((_B,), jnp.float32)),
    mesh=plsc.VectorSubcoreMesh(core_axis_name="c", subcore_axis_name="s"),
    scratch_types=[
        pltpu.VMEM((_NCH, _CH), jnp.int32),
        pltpu.VMEM((_NB, _CH, _D), jnp.float32),
        pltpu.VMEM((_NCH, _CH), jnp.float32),
        pltpu.VMEM((_SPW,), jnp.float32),
        pltpu.SemaphoreType.DMA,
        pltpu.SemaphoreType.DMA,
        pltpu.SemaphoreType.DMA,
    ],
    compiler_params=pltpu.CompilerParams(use_tc_tiling_on_sc=False,
                                         needs_layout_passes=False),
)(_sc_gather_body)


def _tc_body(emb_ref, lin_ref, w1_ref, b1_ref, g1_ref, be1_ref,
             w2_ref, b2_ref, g2_ref, be2_ref, w3_ref, cst_ref, m_ref,
             out_ref, h1_s, h2_s, s1_s, q1_s, s2_s, q2_s, base_s):
    p = pl.program_id(0)
    i = pl.program_id(1)
    row0 = i * _BC

    @pl.when(p == 0)
    def _phase0():
        e = emb_ref[...]
        eb = e.astype(jnp.bfloat16)
        h1 = lax.dot_general(eb, w1_ref[...], (((1,), (1,)), ((), ())),
                             preferred_element_type=jnp.float32)
        h1 = h1 + b1_ref[...]
        h1_s[pl.ds(row0, _BC), :] = h1

        @pl.when(i == 0)
        def _():
            s1_s[...] = jnp.zeros_like(s1_s)
            q1_s[...] = jnp.zeros_like(q1_s)

        s1_s[...] += jnp.sum(h1, axis=0, keepdims=True)
        q1_s[...] += jnp.sum(h1 * h1, axis=0, keepdims=True)

        t = lax.dot_general(eb, m_ref[...], (((1,), (0,)), ((), ())),
                            preferred_element_type=jnp.float32)
        fm = 0.5 * (jnp.sum(t * t, axis=1, keepdims=True)
                    - jnp.sum(e * e, axis=1, keepdims=True))
        base_s[pl.ds(row0, _BC), :] = fm

    @pl.when(p == 1)
    def _phase1():
        mean = s1_s[...] * (1.0 / _B)
        var = q1_s[...] * (1.0 / _B) - mean * mean
        scale = lax.rsqrt(var + _EPS) * g1_ref[...]
        h1 = h1_s[pl.ds(row0, _BC), :]
        a = jnp.maximum((h1 - mean) * scale + be1_ref[...], 0.0)
        h2 = lax.dot_general(a, w2_ref[...], (((1,), (1,)), ((), ())),
                             preferred_element_type=jnp.float32)
        h2 = h2 + b2_ref[...]
        h2_s[pl.ds(row0, _BC), :] = h2

        @pl.when(i == 0)
        def _():
            s2_s[...] = jnp.zeros_like(s2_s)
            q2_s[...] = jnp.zeros_like(q2_s)

        s2_s[...] += jnp.sum(h2, axis=0, keepdims=True)
        q2_s[...] += jnp.sum(h2 * h2, axis=0, keepdims=True)

    @pl.when(p == 2)
    def _phase2():
        mean = s2_s[...] * (1.0 / _B)
        var = q2_s[...] * (1.0 / _B) - mean * mean
        scale = lax.rsqrt(var + _EPS) * g2_ref[...]
        h2 = h2_s[pl.ds(row0, _BC), :]
        a = jnp.maximum((h2 - mean) * scale + be2_ref[...], 0.0)
        mlp = jnp.sum(a * w3_ref[...], axis=1, keepdims=True)
        z = base_s[pl.ds(row0, _BC), :] + lin_ref[...] + mlp + cst_ref[0]
        out_ref[...] = jax.nn.sigmoid(z)


def _tc_forward(e2, lin2, w1, b1, g1, be1, w2, b2, g2, be2, w3, cst, m):
    whole = lambda shape: pl.BlockSpec(shape, lambda p, i: (0,) * len(shape))
    return pl.pallas_call(
        _tc_body,
        grid=(3, _NC),
        in_specs=[
            pl.BlockSpec((_BC, _IN), lambda p, i: (jnp.where(p == 0, i, 0), 0)),
            pl.BlockSpec((_BC, 1), lambda p, i: (jnp.where(p == 2, i, 0), 0)),
            whole((_H1, _IN)),
            whole((1, _H1)),
            whole((1, _H1)),
            whole((1, _H1)),
            whole((_H2, _H1)),
            whole((1, _H2)),
            whole((1, _H2)),
            whole((1, _H2)),
            whole((1, _H2)),
            pl.BlockSpec(memory_space=pltpu.SMEM),
            whole((_IN, _D)),
        ],
        out_specs=pl.BlockSpec((_BC, 1), lambda p, i: (i, 0)),
        out_shape=jax.ShapeDtypeStruct((_B, 1), jnp.float32),
        scratch_shapes=[
            pltpu.VMEM((_B, _H1), jnp.float32),
            pltpu.VMEM((_B, _H2), jnp.float32),
            pltpu.VMEM((1, _H1), jnp.float32),
            pltpu.VMEM((1, _H1), jnp.float32),
            pltpu.VMEM((1, _H2), jnp.float32),
            pltpu.VMEM((1, _H2), jnp.float32),
            pltpu.VMEM((_B, 1), jnp.float32),
        ],
    )(e2, lin2, w1, b1, g1, be1, w2, b2, g2, be2, w3, cst, m)


def kernel(x, offsets, emb, fc_w, fc_b, W1, b1, g1, be1, W2, b2, g2, be2, W3, b3):
    idx = (x + offsets[None, :]).reshape(_NW, _NCH, _CH)
    emb_rows, lin = _sc_gather(idx, emb, lax.reshape(fc_w, (999986,), dimensions=(1, 0)))
    e2 = emb_rows.reshape(_B, _IN)
    lin2 = lin.reshape(_B, 1)
    cst = (fc_b + b3).reshape(1)
    m = jnp.tile(jnp.eye(_D, dtype=jnp.bfloat16), (_F, 1))
    out = _tc_forward(e2, lin2, W1.astype(jnp.bfloat16),
                      b1.reshape(1, _H1), g1.reshape(1, _H1), be1.reshape(1, _H1),
                      W2, b2.reshape(1, _H2), g2.reshape(1, _H2), be2.reshape(1, _H2),
                      W3, cst, m)
    return out.reshape(_B)
